# baseline (speedup 1.0000x reference)
import functools

import jax
import jax.numpy as jnp
import numpy as np
from jax import lax
from jax.experimental import pallas as pl
from jax.experimental.pallas import tpu as pltpu
from jax.experimental.pallas import tpu_sc as plsc

N = 16384
D = 128
LEVELS = 4
NC = 2
NS = 16
NW = NC * NS
ROWS_PER_W = N // NW
CHUNK = 64
NCHUNK = ROWS_PER_W // CHUNK
LANES = 16
NSETS = 3
GROWS = LEVELS * CHUNK


def _body(idx_hbm, table, out, idx_v, b0, b1, b2, o0, o1, o2, sems):
    wid = lax.axis_index("s") * NC + lax.axis_index("c")
    base = wid * ROWS_PER_W
    bufs = (b0, b1, b2)
    obufs = (o0, o1, o2)
    gsems, wsems = sems[:NSETS], sems[NSETS:]

    pltpu.sync_copy(idx_hbm.at[wid], idx_v)

    def fire_gathers(k, s):
        return [
            pltpu.async_copy(table.at[idx_v.at[2 * k + j]],
                             bufs[s].at[pl.ds(j * 2 * CHUNK, 2 * CHUNK)],
                             gsems[s])
            for j in range(2)
        ]

    gcps = [fire_gathers(0, 0), fire_gathers(1, 1), None]
    wcps = [None] * NSETS
    for k in range(NCHUNK):
        s = k % NSETS
        if k + 2 < NCHUNK:
            gcps[(k + 2) % NSETS] = fire_gathers(k + 2, (k + 2) % NSETS)
        with jax.named_scope("gwait"):
            for cp in gcps[s]:
                cp.wait()
        with jax.named_scope("wbwait"):
            if wcps[s] is not None:
                wcps[s].wait()
        b, o = bufs[s], obufs[s]

        def add_row(r, _, b=b, o=o):
            for c in range(D // 32):
                lo = hi = None
                for l in range(LEVELS):
                    w = b[r + l * CHUNK, pl.ds(c * LANES, LANES)]
                    wlo = lax.bitcast_convert_type(
                        lax.shift_left(w, 16), jnp.float32)
                    whi = lax.bitcast_convert_type(
                        lax.bitwise_and(w, jnp.int32(-65536)), jnp.float32)
                    lo = wlo if lo is None else lo + wlo
                    hi = whi if hi is None else hi + whi
                o[r, pl.ds(c * 32, LANES)] = lo
                o[r, pl.ds(c * 32 + LANES, LANES)] = hi
            return 0

        with jax.named_scope("adds"):
            lax.fori_loop(0, CHUNK, add_row, 0)
        wcps[s] = pltpu.async_copy(
            o, out.at[pl.ds(base + k * CHUNK, CHUNK)], wsems[s])
    for cp in wcps:
        if cp is not None:
            cp.wait()


def _entry(idx_hbm, table, out, idx_v, b0, b1, b2, o0, o1, o2, *sems):
    _body(idx_hbm, table, out, idx_v, b0, b1, b2, o0, o1, o2, sems)


_mesh = plsc.VectorSubcoreMesh(core_axis_name="c", subcore_axis_name="s")

_sc_call = functools.partial(
    pl.kernel,
    mesh=_mesh,
    compiler_params=pltpu.CompilerParams(use_tc_tiling_on_sc=False),
    out_type=jax.ShapeDtypeStruct((N, D), jnp.float32),
    scratch_types=(
        [pltpu.VMEM((2 * NCHUNK, 2 * CHUNK), jnp.int32)]
        + [pltpu.VMEM((GROWS, D // 2), jnp.int32)] * NSETS
        + [pltpu.VMEM((CHUNK, D), jnp.float32)] * NSETS
        + [pltpu.SemaphoreType.DMA] * (2 * NSETS)
    ),
)(_entry)


@jax.jit
def kernel(coords, emb0, emb1, emb2, emb3):
    perm = np.empty((D,), np.int32)
    for c in range(D // 32):
        for i in range(16):
            perm[32 * c + 2 * i] = 32 * c + i
            perm[32 * c + 2 * i + 1] = 32 * c + 16 + i
    table = jnp.concatenate([emb0, emb1, emb2, emb3], axis=0)
    table = table.astype(jnp.bfloat16)[:, perm]
    table = jax.lax.bitcast_convert_type(
        table.reshape(LEVELS * emb0.shape[0], D // 2, 2), jnp.int32)
    off = jnp.arange(LEVELS, dtype=jnp.int32) * emb0.shape[0]
    idx = (coords.reshape(NW, NCHUNK, CHUNK, LEVELS).transpose(0, 1, 3, 2)
           + off[None, None, :, None])
    idx = idx.reshape(NW, 2 * NCHUNK, 2 * CHUNK)
    return _sc_call(idx, table)


# speedup vs baseline: 1.1586x; 1.1586x over previous
import functools

import jax
import jax.numpy as jnp
from jax import lax
from jax.experimental import pallas as pl
from jax.experimental.pallas import tpu as pltpu
from jax.experimental.pallas import tpu_sc as plsc

N = 16384
D = 128
LEVELS = 4
NC = 2
NS = 16
NW = NC * NS
ROWS_PER_W = N // NW
CHUNK = 64
NCHUNK = ROWS_PER_W // CHUNK
LANES = 16
NSETS = 3
GROWS = LEVELS * CHUNK


def _body(idx_hbm, table, out, idx_v, b0, b1, b2, o0, o1, o2, sems):
    wid = lax.axis_index("s") * NC + lax.axis_index("c")
    base = wid * ROWS_PER_W
    bufs = (b0, b1, b2)
    obufs = (o0, o1, o2)
    gsems, wsems = sems[:NSETS], sems[NSETS:]

    pltpu.sync_copy(idx_hbm.at[wid], idx_v)

    def fire_gathers(k, s):
        return [
            pltpu.async_copy(table.at[idx_v.at[2 * k + j]],
                             bufs[s].at[pl.ds(j * 2 * CHUNK, 2 * CHUNK)],
                             gsems[s])
            for j in range(2)
        ]

    gcps = [fire_gathers(0, 0), fire_gathers(1, 1), None]
    wcps = [None] * NSETS
    for k in range(NCHUNK):
        s = k % NSETS
        if k + 2 < NCHUNK:
            gcps[(k + 2) % NSETS] = fire_gathers(k + 2, (k + 2) % NSETS)
        with jax.named_scope("gwait"):
            for cp in gcps[s]:
                cp.wait()
        with jax.named_scope("wbwait"):
            if wcps[s] is not None:
                wcps[s].wait()
        b, o = bufs[s], obufs[s]

        def add_row(r, _, b=b, o=o):
            for c in range(D // 32):
                lo = hi = None
                for l in range(LEVELS):
                    w = b[r + l * CHUNK, pl.ds(c * LANES, LANES)]
                    wlo = lax.bitcast_convert_type(
                        lax.shift_left(w, 16), jnp.float32)
                    whi = lax.bitcast_convert_type(w, jnp.float32)
                    lo = wlo if lo is None else lo + wlo
                    hi = whi if hi is None else hi + whi
                o[r, pl.ds(c * 32, LANES)] = lo
                o[r, pl.ds(c * 32 + LANES, LANES)] = hi
            return 0

        with jax.named_scope("adds"):
            lax.fori_loop(0, CHUNK, add_row, 0)
        wcps[s] = pltpu.async_copy(
            o, out.at[pl.ds(base + k * CHUNK, CHUNK)], wsems[s])
    for cp in wcps:
        if cp is not None:
            cp.wait()


def _entry(idx_hbm, table, out, idx_v, b0, b1, b2, o0, o1, o2, *sems):
    _body(idx_hbm, table, out, idx_v, b0, b1, b2, o0, o1, o2, sems)


_mesh = plsc.VectorSubcoreMesh(core_axis_name="c", subcore_axis_name="s")

_sc_call = functools.partial(
    pl.kernel,
    mesh=_mesh,
    compiler_params=pltpu.CompilerParams(use_tc_tiling_on_sc=False),
    out_type=jax.ShapeDtypeStruct((N, D), jnp.float32),
    scratch_types=(
        [pltpu.VMEM((2 * NCHUNK, 2 * CHUNK), jnp.int32)]
        + [pltpu.VMEM((GROWS, D // 2), jnp.int32)] * NSETS
        + [pltpu.VMEM((CHUNK, D), jnp.float32)] * NSETS
        + [pltpu.SemaphoreType.DMA] * (2 * NSETS)
    ),
)(_entry)


@jax.jit
def kernel(coords, emb0, emb1, emb2, emb3):
    v = LEVELS * emb0.shape[0]
    table = jnp.concatenate([emb0, emb1, emb2, emb3], axis=0)
    u = lax.bitcast_convert_type(table.astype(jnp.bfloat16),
                                 jnp.uint16).astype(jnp.uint32)
    u = u.reshape(v, D // 32, 32)
    w = u[:, :, :LANES] | (u[:, :, LANES:] << 16)
    table = lax.bitcast_convert_type(w, jnp.int32).reshape(v, D // 2)
    off = jnp.arange(LEVELS, dtype=jnp.int32) * emb0.shape[0]
    idx = (coords.reshape(NW, NCHUNK, CHUNK, LEVELS).transpose(0, 1, 3, 2)
           + off[None, None, :, None])
    idx = idx.reshape(NW, 2 * NCHUNK, 2 * CHUNK)
    return _sc_call(idx, table)


# speedup vs baseline: 1.2009x; 1.0365x over previous
import functools

import jax
import jax.numpy as jnp
from jax import lax
from jax.experimental import pallas as pl
from jax.experimental.pallas import tpu as pltpu
from jax.experimental.pallas import tpu_sc as plsc

N = 16384
D = 128
LEVELS = 4
NC = 2
NS = 16
NW = NC * NS
ROWS_PER_W = N // NW
CHUNK = 64
NCHUNK = ROWS_PER_W // CHUNK
LANES = 16
NSETS = 3


def _body(idx_hbm, t01, t23, out, idx_v, f0, f1, f2, i0, i1, i2,
          o0, o1, o2, sems):
    wid = lax.axis_index("s") * NC + lax.axis_index("c")
    base = wid * ROWS_PER_W
    fbufs = (f0, f1, f2)
    ibufs = (i0, i1, i2)
    obufs = (o0, o1, o2)
    gsems, wsems = sems[:NSETS], sems[NSETS:]

    pltpu.sync_copy(idx_hbm.at[wid], idx_v)

    def fire_gathers(k, s):
        return [
            pltpu.async_copy(t01.at[idx_v.at[2 * k]], fbufs[s], gsems[s]),
            pltpu.async_copy(t23.at[idx_v.at[2 * k + 1]], ibufs[s], gsems[s]),
        ]

    gcps = [fire_gathers(0, 0), fire_gathers(1, 1), None]
    wcps = [None] * NSETS
    for k in range(NCHUNK):
        s = k % NSETS
        if k + 2 < NCHUNK:
            gcps[(k + 2) % NSETS] = fire_gathers(k + 2, (k + 2) % NSETS)
        with jax.named_scope("gwait"):
            for cp in gcps[s]:
                cp.wait()
        with jax.named_scope("wbwait"):
            if wcps[s] is not None:
                wcps[s].wait()
        bf, bi, o = fbufs[s], ibufs[s], obufs[s]

        def add_row(r, _, bf=bf, bi=bi, o=o):
            for c in range(D // 32):
                w2 = bi[r, pl.ds(c * LANES, LANES)]
                w3 = bi[r + CHUNK, pl.ds(c * LANES, LANES)]
                lo = (bf[r, pl.ds(c * 32, LANES)]
                      + bf[r + CHUNK, pl.ds(c * 32, LANES)]
                      + lax.bitcast_convert_type(
                          lax.shift_left(w2, 16), jnp.float32)
                      + lax.bitcast_convert_type(
                          lax.shift_left(w3, 16), jnp.float32))
                hi = (bf[r, pl.ds(c * 32 + LANES, LANES)]
                      + bf[r + CHUNK, pl.ds(c * 32 + LANES, LANES)]
                      + lax.bitcast_convert_type(w2, jnp.float32)
                      + lax.bitcast_convert_type(w3, jnp.float32))
                o[r, pl.ds(c * 32, LANES)] = lo
                o[r, pl.ds(c * 32 + LANES, LANES)] = hi
            return 0

        with jax.named_scope("adds"):
            lax.fori_loop(0, CHUNK, add_row, 0)
        wcps[s] = pltpu.async_copy(
            o, out.at[pl.ds(base + k * CHUNK, CHUNK)], wsems[s])
    for cp in wcps:
        if cp is not None:
            cp.wait()


def _entry(idx_hbm, t01, t23, out, idx_v, f0, f1, f2, i0, i1, i2,
           o0, o1, o2, *sems):
    _body(idx_hbm, t01, t23, out, idx_v, f0, f1, f2, i0, i1, i2,
          o0, o1, o2, sems)


_mesh = plsc.VectorSubcoreMesh(core_axis_name="c", subcore_axis_name="s")

_sc_call = functools.partial(
    pl.kernel,
    mesh=_mesh,
    compiler_params=pltpu.CompilerParams(use_tc_tiling_on_sc=False),
    out_type=jax.ShapeDtypeStruct((N, D), jnp.float32),
    scratch_types=(
        [pltpu.VMEM((2 * NCHUNK, 2 * CHUNK), jnp.int32)]
        + [pltpu.VMEM((2 * CHUNK, D), jnp.float32)] * NSETS
        + [pltpu.VMEM((2 * CHUNK, D // 2), jnp.int32)] * NSETS
        + [pltpu.VMEM((CHUNK, D), jnp.float32)] * NSETS
        + [pltpu.SemaphoreType.DMA] * (2 * NSETS)
    ),
)(_entry)


@jax.jit
def kernel(coords, emb0, emb1, emb2, emb3):
    v = 2 * emb0.shape[0]
    t01 = jnp.concatenate([emb0, emb1], axis=0)
    u = lax.bitcast_convert_type(
        jnp.concatenate([emb2, emb3], axis=0).astype(jnp.bfloat16),
        jnp.uint16).astype(jnp.uint32)
    u = u.reshape(v, D // 32, 32)
    w = u[:, :, :LANES] | (u[:, :, LANES:] << 16)
    t23 = lax.bitcast_convert_type(w, jnp.int32).reshape(v, D // 2)
    off = jnp.array([0, 1, 0, 1], jnp.int32) * emb0.shape[0]
    idx = (coords.reshape(NW, NCHUNK, CHUNK, LEVELS).transpose(0, 1, 3, 2)
           + off[None, None, :, None])
    idx = idx.reshape(NW, 2 * NCHUNK, 2 * CHUNK)
    return _sc_call(idx, t01, t23)


# speedup vs baseline: 1.2635x; 1.0521x over previous
import functools

import jax
import jax.numpy as jnp
from jax import lax
from jax.experimental import pallas as pl
from jax.experimental.pallas import tpu as pltpu
from jax.experimental.pallas import tpu_sc as plsc

N = 16384
D = 128
LEVELS = 4
NC = 2
NS = 16
NW = NC * NS
ROWS_PER_W = N // NW
CHUNK = 64
NCHUNK = ROWS_PER_W // CHUNK
LANES = 16
NSETS = 3
GROWS = LEVELS * CHUNK


def _body(idx_hbm, table, out, idx_v, b0, b1, b2, o0, o1, o2, sems):
    wid = lax.axis_index("s") * NC + lax.axis_index("c")
    base = wid * ROWS_PER_W
    bufs = (b0, b1, b2)
    obufs = (o0, o1, o2)
    gsems, wsems = sems[:NSETS], sems[NSETS:]

    pltpu.sync_copy(idx_hbm.at[wid], idx_v)

    def fire_gathers(k, s):
        return [
            pltpu.async_copy(table.at[idx_v.at[2 * k + j]],
                             bufs[s].at[pl.ds(j * 2 * CHUNK, 2 * CHUNK)],
                             gsems[s])
            for j in range(2)
        ]

    gcps = [fire_gathers(0, 0), fire_gathers(1, 1), None]
    wcps = [None] * NSETS
    for k in range(NCHUNK):
        s = k % NSETS
        if k + 2 < NCHUNK:
            gcps[(k + 2) % NSETS] = fire_gathers(k + 2, (k + 2) % NSETS)
        with jax.named_scope("gwait"):
            for cp in gcps[s]:
                cp.wait()
        with jax.named_scope("wbwait"):
            if wcps[s] is not None:
                wcps[s].wait()
        b, o = bufs[s], obufs[s]

        def add_row(r, _, b=b, o=o):
            for col in range(D // LANES):
                sl = pl.ds(col * LANES, LANES)
                o[r, sl] = (b[r, sl] + b[r + CHUNK, sl]
                            + b[r + 2 * CHUNK, sl] + b[r + 3 * CHUNK, sl])
            return 0

        with jax.named_scope("adds"):
            lax.fori_loop(0, CHUNK, add_row, 0)
        wcps[s] = pltpu.async_copy(
            o, out.at[pl.ds(base + k * CHUNK, CHUNK)], wsems[s])
    for cp in wcps:
        if cp is not None:
            cp.wait()


def _entry(idx_hbm, table, out, idx_v, b0, b1, b2, o0, o1, o2, *sems):
    _body(idx_hbm, table, out, idx_v, b0, b1, b2, o0, o1, o2, sems)


_mesh = plsc.VectorSubcoreMesh(core_axis_name="c", subcore_axis_name="s")

_sc_call = functools.partial(
    pl.kernel,
    mesh=_mesh,
    out_type=jax.ShapeDtypeStruct((N, D), jnp.float32),
    scratch_types=(
        [pltpu.VMEM((2 * NCHUNK, 2 * CHUNK), jnp.int32)]
        + [pltpu.VMEM((GROWS, D), jnp.float32)] * NSETS
        + [pltpu.VMEM((CHUNK, D), jnp.float32)] * NSETS
        + [pltpu.SemaphoreType.DMA] * (2 * NSETS)
    ),
)(_entry)


@jax.jit
def kernel(coords, emb0, emb1, emb2, emb3):
    table = jnp.concatenate([emb0, emb1, emb2, emb3], axis=0)
    off = jnp.arange(LEVELS, dtype=jnp.int32) * emb0.shape[0]
    idx = (coords.reshape(NW, NCHUNK, CHUNK, LEVELS).transpose(0, 1, 3, 2)
           + off[None, None, :, None])
    idx = idx.reshape(NW, 2 * NCHUNK, 2 * CHUNK)
    return _sc_call(idx, table)
